# Initial kernel scaffold; baseline (speedup 1.0000x reference)
import functools

import jax
import jax.numpy as jnp
from jax import lax
from jax.experimental import pallas as pl
from jax.experimental.pallas import tpu as pltpu
from jax.experimental.pallas import tpu_sc as plsc

N_NODES = 100000
N_EDGES = 6400000
SHORT_CUTOFF = 4.0
LONG_CUTOFF = 12.0
KEHALF = 7.199822675975274

NC, NS, L = 2, 16, 16
NW = NC * NS
EPW = N_EDGES // NW
CHUNK = 2000
NCHUNK = EPW // CHUNK
GROUPS = CHUNK // L
SLICE = 6400
PAD = NS * SLICE

_mesh = plsc.VectorSubcoreMesh(
    core_axis_name="c", subcore_axis_name="s", num_cores=NC, num_subcores=NS
)


@functools.partial(
    pl.kernel,
    out_type=jax.ShapeDtypeStruct((NC, PAD), jnp.float32),
    mesh=_mesh,
    scratch_types=[
        pltpu.VMEM((N_NODES,), jnp.float32),
        pltpu.VMEM((CHUNK,), jnp.int32),
        pltpu.VMEM((CHUNK,), jnp.int32),
        pltpu.VMEM((CHUNK,), jnp.float32),
        pltpu.VMEM((CHUNK,), jnp.float32),
        pltpu.VMEM((SLICE,), jnp.float32),
        pltpu.VMEM_SHARED((PAD,), jnp.float32),
    ],
)
def _sc_energy(q_hbm, d_hbm, ii_hbm, jj_hbm, out_hbm,
               table_v, ii_v, jj_v, dd_v, ee_v, stage_v, acc_s):
    cid = lax.axis_index("c")
    sid = lax.axis_index("s")
    wid = sid * NC + cid

    def _zero(k, carry):
        stage_v[pl.ds(k * L, L)] = jnp.zeros((L,), jnp.float32)
        return carry

    lax.fori_loop(0, SLICE // L, _zero, 0)
    pltpu.sync_copy(stage_v, acc_s.at[pl.ds(sid * SLICE, SLICE)])

    pltpu.sync_copy(q_hbm, table_v)
    plsc.subcore_barrier()

    lr2_inv = 1.0 / (LONG_CUTOFF * LONG_CUTOFF)
    shift = 2.0 / LONG_CUTOFF

    def _chunk(g, carry):
        base = wid * EPW + g * CHUNK
        pltpu.sync_copy(ii_hbm.at[pl.ds(base, CHUNK)], ii_v)
        pltpu.sync_copy(jj_hbm.at[pl.ds(base, CHUNK)], jj_v)
        pltpu.sync_copy(d_hbm.at[pl.ds(base, CHUNK)], dd_v)

        def _grp(k, c2):
            s = k * L
            ii = ii_v[pl.ds(s, L)]
            jj = jj_v[pl.ds(s, L)]
            d = dd_v[pl.ds(s, L)]
            qi = plsc.load_gather(table_v, [ii])
            qj = plsc.load_gather(table_v, [jj])

            s2 = d * d + 1.0
            yi = 0x5F3759DF - (plsc.bitcast(s2, jnp.int32) >> 1)
            y = plsc.bitcast(yi, jnp.float32)
            h = 0.5 * s2
            y = y * (1.5 - h * y * y)
            y = y * (1.5 - h * y * y)
            y = y * (1.5 - h * y * y)
            dsh = s2 * y

            inv_d = 1.0 / d
            e_ord = inv_d + d * lr2_inv - shift
            e_shl = y + dsh * lr2_inv - shift

            x = d * (1.0 / SHORT_CUTOFF)
            x3 = x * x * x
            sw = 1.0 + x3 * (-10.0 + x * (15.0 - 6.0 * x))
            sw = jnp.where(d < SHORT_CUTOFF, sw, jnp.zeros_like(sw))

            e = KEHALF * qi * qj * (sw * e_ord + (1.0 - sw) * e_shl)
            e = jnp.where(d <= LONG_CUTOFF, e, jnp.zeros_like(e))
            ee_v[pl.ds(s, L)] = e
            return c2

        lax.fori_loop(0, GROUPS, _grp, 0)
        pltpu.sync_copy(ee_v, acc_s.at[ii_v], add=True)
        return carry

    lax.fori_loop(0, NCHUNK, _chunk, 0)
    plsc.subcore_barrier()

    pltpu.sync_copy(acc_s.at[pl.ds(sid * SLICE, SLICE)], stage_v)
    pltpu.sync_copy(stage_v, out_hbm.at[cid, pl.ds(sid * SLICE, SLICE)])


def kernel(atomic_charges, distances, idx_i, idx_j):
    parts = _sc_energy(atomic_charges, distances, idx_i, idx_j)
    return parts[0, :N_NODES] + parts[1, :N_NODES]


# speedup vs baseline: 258.6062x; 258.6062x over previous
import functools

import jax
import jax.numpy as jnp
from jax import lax
from jax.experimental import pallas as pl
from jax.experimental.pallas import tpu as pltpu
from jax.experimental.pallas import tpu_sc as plsc

N_NODES = 100000
N_EDGES = 6400000
SHORT_CUTOFF = 4.0
LONG_CUTOFF = 12.0
KEHALF = 7.199822675975274

NC, NS, L = 2, 16, 16
NW = NC * NS
EPW = N_EDGES // NW
CHUNK = 2000
NCHUNK = EPW // CHUNK
GROUPS = CHUNK // L
SLICE = 6400
PAD = NS * SLICE

_mesh = plsc.VectorSubcoreMesh(
    core_axis_name="c", subcore_axis_name="s", num_cores=NC, num_subcores=NS
)


@functools.partial(
    pl.kernel,
    out_type=jax.ShapeDtypeStruct((NC, PAD), jnp.float32),
    mesh=_mesh,
    scratch_types=[
        pltpu.VMEM((N_NODES,), jnp.float32),
        pltpu.VMEM((CHUNK,), jnp.int32),
        pltpu.VMEM((CHUNK,), jnp.int32),
        pltpu.VMEM((CHUNK,), jnp.float32),
        pltpu.VMEM((CHUNK,), jnp.float32),
        pltpu.VMEM((SLICE,), jnp.float32),
        pltpu.VMEM_SHARED((PAD,), jnp.float32),
    ],
    compiler_params=pltpu.CompilerParams(needs_layout_passes=False),
)
def _sc_energy(q_hbm, d_hbm, ii_hbm, jj_hbm, out_hbm,
               table_v, ii_v, jj_v, dd_v, ee_v, stage_v, acc_s):
    cid = lax.axis_index("c")
    sid = lax.axis_index("s")
    wid = sid * NC + cid

    def _zero(k, carry):
        stage_v[pl.ds(k * L, L)] = jnp.zeros((L,), jnp.float32)
        return carry

    lax.fori_loop(0, SLICE // L, _zero, 0)
    pltpu.sync_copy(stage_v, acc_s.at[pl.ds(sid * SLICE, SLICE)])

    pltpu.sync_copy(q_hbm, table_v)
    plsc.subcore_barrier()

    lr2_inv = 1.0 / (LONG_CUTOFF * LONG_CUTOFF)
    shift = 2.0 / LONG_CUTOFF

    def _chunk(g, carry):
        base = wid * EPW + g * CHUNK
        pltpu.sync_copy(ii_hbm.at[pl.ds(base, CHUNK)], ii_v)
        pltpu.sync_copy(jj_hbm.at[pl.ds(base, CHUNK)], jj_v)
        pltpu.sync_copy(d_hbm.at[pl.ds(base, CHUNK)], dd_v)

        def _grp(k, c2):
            s = k * L
            ii = ii_v[pl.ds(s, L)]
            jj = jj_v[pl.ds(s, L)]
            d = dd_v[pl.ds(s, L)]
            qi = plsc.load_gather(table_v, [ii])
            qj = plsc.load_gather(table_v, [jj])

            s2 = d * d + 1.0
            yi = 0x5F3759DF - (plsc.bitcast(s2, jnp.int32) >> 1)
            y = plsc.bitcast(yi, jnp.float32)
            h = 0.5 * s2
            y = y * (1.5 - h * y * y)
            y = y * (1.5 - h * y * y)
            y = y * (1.5 - h * y * y)
            dsh = s2 * y

            inv_d = 1.0 / d
            e_ord = inv_d + d * lr2_inv - shift
            e_shl = y + dsh * lr2_inv - shift

            x = d * (1.0 / SHORT_CUTOFF)
            x3 = x * x * x
            sw = 1.0 + x3 * (-10.0 + x * (15.0 - 6.0 * x))
            sw = jnp.where(d < SHORT_CUTOFF, sw, jnp.zeros_like(sw))

            e = KEHALF * qi * qj * (sw * e_ord + (1.0 - sw) * e_shl)
            e = jnp.where(d <= LONG_CUTOFF, e, jnp.zeros_like(e))
            ee_v[pl.ds(s, L)] = e
            return c2

        lax.fori_loop(0, GROUPS, _grp, 0)
        pltpu.sync_copy(ee_v, acc_s.at[ii_v], add=True)
        return carry

    lax.fori_loop(0, NCHUNK, _chunk, 0)
    plsc.subcore_barrier()

    pltpu.sync_copy(acc_s.at[pl.ds(sid * SLICE, SLICE)], stage_v)
    pltpu.sync_copy(stage_v, out_hbm.at[cid, pl.ds(sid * SLICE, SLICE)])


def kernel(atomic_charges, distances, idx_i, idx_j):
    parts = _sc_energy(atomic_charges, distances, idx_i, idx_j)
    return parts[0, :N_NODES] + parts[1, :N_NODES]


# speedup vs baseline: 259.4418x; 1.0032x over previous
import functools

import jax
import jax.numpy as jnp
from jax import lax
from jax.experimental import pallas as pl
from jax.experimental.pallas import tpu as pltpu
from jax.experimental.pallas import tpu_sc as plsc

N_NODES = 100000
N_EDGES = 6400000
SHORT_CUTOFF = 4.0
LONG_CUTOFF = 12.0
KEHALF = 7.199822675975274

NC, NS, L = 2, 16, 16
NW = NC * NS
EPW = N_EDGES // NW
CHUNK = 2000
NCHUNK = EPW // CHUNK
GROUPS = CHUNK // L
SLICE = 6400
PAD = NS * SLICE

_mesh = plsc.VectorSubcoreMesh(
    core_axis_name="c", subcore_axis_name="s", num_cores=NC, num_subcores=NS
)


@functools.partial(
    pl.kernel,
    out_type=jax.ShapeDtypeStruct((NC, PAD), jnp.float32),
    mesh=_mesh,
    scratch_types=[
        pltpu.VMEM((N_NODES,), jnp.float32),
        pltpu.VMEM((CHUNK,), jnp.int32),
        pltpu.VMEM((CHUNK,), jnp.int32),
        pltpu.VMEM((CHUNK,), jnp.float32),
        pltpu.VMEM((CHUNK,), jnp.float32),
        pltpu.VMEM((SLICE,), jnp.float32),
        pltpu.VMEM_SHARED((PAD,), jnp.float32),
    ],
    compiler_params=pltpu.CompilerParams(needs_layout_passes=False),
)
def _sc_energy(q_hbm, d_hbm, ii_hbm, jj_hbm, out_hbm,
               table_v, ii_v, jj_v, dd_v, ee_v, stage_v, acc_s):
    cid = lax.axis_index("c")
    sid = lax.axis_index("s")
    wid = sid * NC + cid

    def _zero(k, carry):
        stage_v[pl.ds(k * L, L)] = jnp.zeros((L,), jnp.float32)
        return carry

    lax.fori_loop(0, SLICE // L, _zero, 0)
    pltpu.sync_copy(stage_v, acc_s.at[pl.ds(sid * SLICE, SLICE)])

    pltpu.sync_copy(q_hbm, table_v)
    plsc.subcore_barrier()

    lr2_inv = 1.0 / (LONG_CUTOFF * LONG_CUTOFF)
    shift = 2.0 / LONG_CUTOFF

    def _chunk(g, carry):
        base = wid * EPW + g * CHUNK
        pltpu.sync_copy(ii_hbm.at[pl.ds(base, CHUNK)], ii_v)
        pltpu.sync_copy(jj_hbm.at[pl.ds(base, CHUNK)], jj_v)
        pltpu.sync_copy(d_hbm.at[pl.ds(base, CHUNK)], dd_v)

        @plsc.parallel_loop(0, CHUNK, L, unroll=4)
        def _grp(s):
            ii = ii_v[pl.ds(s, L)]
            jj = jj_v[pl.ds(s, L)]
            d = dd_v[pl.ds(s, L)]
            qi = plsc.load_gather(table_v, [ii])
            qj = plsc.load_gather(table_v, [jj])

            s2 = d * d + 1.0
            yi = 0x5F3759DF - (plsc.bitcast(s2, jnp.int32) >> 1)
            y = plsc.bitcast(yi, jnp.float32)
            h = 0.5 * s2
            y = y * (1.5 - h * y * y)
            y = y * (1.5 - h * y * y)
            dsh = s2 * y

            r = plsc.bitcast(0x7EF311C2 - plsc.bitcast(d, jnp.int32),
                             jnp.float32)
            r = r * (2.0 - d * r)
            r = r * (2.0 - d * r)

            e_shl = y + dsh * lr2_inv - shift
            diff = (r - y) + (d - dsh) * lr2_inv

            x = d * (1.0 / SHORT_CUTOFF)
            x3 = x * x * x
            sw = 1.0 + x3 * (-10.0 + x * (15.0 - 6.0 * x))
            sw = jnp.where(d < SHORT_CUTOFF, sw, jnp.zeros_like(sw))

            e = (KEHALF * qi) * qj * (e_shl + sw * diff)
            e = jnp.where(d <= LONG_CUTOFF, e, jnp.zeros_like(e))
            ee_v[pl.ds(s, L)] = e
        pltpu.sync_copy(ee_v, acc_s.at[ii_v], add=True)
        return carry

    lax.fori_loop(0, NCHUNK, _chunk, 0)
    plsc.subcore_barrier()

    pltpu.sync_copy(acc_s.at[pl.ds(sid * SLICE, SLICE)], stage_v)
    pltpu.sync_copy(stage_v, out_hbm.at[cid, pl.ds(sid * SLICE, SLICE)])


def kernel(atomic_charges, distances, idx_i, idx_j):
    parts = _sc_energy(atomic_charges, distances, idx_i, idx_j)
    return parts[0, :N_NODES] + parts[1, :N_NODES]


# speedup vs baseline: 614.4237x; 2.3683x over previous
import functools

import jax
import jax.numpy as jnp
from jax import lax
from jax.experimental import pallas as pl
from jax.experimental.pallas import tpu as pltpu
from jax.experimental.pallas import tpu_sc as plsc

N_NODES = 100000
N_EDGES = 6400000
SHORT_CUTOFF = 4.0
LONG_CUTOFF = 12.0
KEHALF = 7.199822675975274

NC, NS, L = 2, 16, 16
NW = NC * NS
EPW = N_EDGES // NW
CHUNK = 800
NCHUNK = EPW // CHUNK
SLICE = 6400
PAD = NS * SLICE

_mesh = plsc.VectorSubcoreMesh(
    core_axis_name="c", subcore_axis_name="s", num_cores=NC, num_subcores=NS
)


@functools.partial(
    pl.kernel,
    out_type=jax.ShapeDtypeStruct((NC, PAD), jnp.float32),
    mesh=_mesh,
    scratch_types=[
        pltpu.VMEM((N_NODES,), jnp.float32),
        pltpu.VMEM((CHUNK,), jnp.int32),
        pltpu.VMEM((CHUNK,), jnp.int32),
        pltpu.VMEM((CHUNK,), jnp.int32),
        pltpu.VMEM((CHUNK,), jnp.int32),
        pltpu.VMEM((CHUNK,), jnp.float32),
        pltpu.VMEM((CHUNK,), jnp.float32),
        pltpu.VMEM((CHUNK,), jnp.float32),
        pltpu.VMEM((CHUNK,), jnp.float32),
        pltpu.VMEM((CHUNK,), jnp.int32),
        pltpu.VMEM((CHUNK,), jnp.int32),
        pltpu.VMEM((SLICE,), jnp.float32),
        pltpu.VMEM_SHARED((PAD,), jnp.float32),
        pltpu.SemaphoreType.DMA,
        pltpu.SemaphoreType.DMA,
        pltpu.SemaphoreType.DMA,
        pltpu.SemaphoreType.DMA,
    ],
    compiler_params=pltpu.CompilerParams(needs_layout_passes=False),
)
def _sc_energy(q_hbm, d_hbm, ii_hbm, jj_hbm, out_hbm,
               table_v, ii0, ii1, jj0, jj1, dd0, dd1, ee0, ee1, iis0, iis1,
               stage_v, acc_s, sin0, sin1, ssc0, ssc1):
    cid = lax.axis_index("c")
    sid = lax.axis_index("s")
    wid = sid * NC + cid
    ii_v, jj_v, dd_v = (ii0, ii1), (jj0, jj1), (dd0, dd1)
    ee_v, iis_v = (ee0, ee1), (iis0, iis1)
    sins = (sin0, sin1)
    sscs = (ssc0, ssc1)

    def _zero(k, carry):
        stage_v[pl.ds(k * L, L)] = jnp.zeros((L,), jnp.float32)
        return carry

    lax.fori_loop(0, SLICE // L, _zero, 0)
    pltpu.sync_copy(stage_v, acc_s.at[pl.ds(sid * SLICE, SLICE)])

    pltpu.sync_copy(q_hbm, table_v)
    plsc.subcore_barrier()

    lr2_inv = 1.0 / (LONG_CUTOFF * LONG_CUTOFF)
    shift = 2.0 / LONG_CUTOFF

    def start_in(g, b):
        base = wid * EPW + g * CHUNK
        pltpu.async_copy(ii_hbm.at[pl.ds(base, CHUNK)], ii_v[b], sins[b])
        pltpu.async_copy(jj_hbm.at[pl.ds(base, CHUNK)], jj_v[b], sins[b])
        pltpu.async_copy(d_hbm.at[pl.ds(base, CHUNK)], dd_v[b], sins[b])

    def wait_in(b):
        pltpu.make_async_copy(ii_hbm.at[pl.ds(0, CHUNK)], ii_v[b], sins[b]).wait()
        pltpu.make_async_copy(jj_hbm.at[pl.ds(0, CHUNK)], jj_v[b], sins[b]).wait()
        pltpu.make_async_copy(d_hbm.at[pl.ds(0, CHUNK)], dd_v[b], sins[b]).wait()

    def start_sc(b):
        pltpu.async_copy(ee_v[b], acc_s.at[iis_v[b]], sscs[b], add=True)

    def wait_sc(b):
        pltpu.make_async_copy(ee_v[b], acc_s.at[iis_v[b]], sscs[b]).wait()

    start_in(0, 0)

    def _outer(t, carry):
        for b in range(2):
            g = 2 * t + b
            wait_in(b)

            @pl.when(g + 1 < NCHUNK)
            def _():
                start_in(g + 1, 1 - b)


            @pl.when(g >= 2)
            def _():
                wait_sc(b)

            @plsc.parallel_loop(0, CHUNK, L, unroll=4)
            def _grp(s):
                ii = ii_v[b][pl.ds(s, L)]
                jj = jj_v[b][pl.ds(s, L)]
                d = dd_v[b][pl.ds(s, L)]
                qi = plsc.load_gather(table_v, [ii])
                qj = plsc.load_gather(table_v, [jj])

                s2 = d * d + 1.0
                yi = 0x5F3759DF - (plsc.bitcast(s2, jnp.int32) >> 1)
                y = plsc.bitcast(yi, jnp.float32)
                h = 0.5 * s2
                y = y * (1.5 - h * y * y)
                y = y * (1.5 - h * y * y)
                dsh = s2 * y

                r = plsc.bitcast(0x7EF311C2 - plsc.bitcast(d, jnp.int32),
                                 jnp.float32)
                r = r * (2.0 - d * r)
                r = r * (2.0 - d * r)

                e_shl = y + dsh * lr2_inv - shift
                diff = (r - y) + (d - dsh) * lr2_inv

                x = d * (1.0 / SHORT_CUTOFF)
                x3 = x * x * x
                sw = 1.0 + x3 * (-10.0 + x * (15.0 - 6.0 * x))
                sw = jnp.where(d < SHORT_CUTOFF, sw, jnp.zeros_like(sw))

                e = (KEHALF * qi) * qj * (e_shl + sw * diff)
                e = jnp.where(d <= LONG_CUTOFF, e, jnp.zeros_like(e))
                ee_v[b][pl.ds(s, L)] = e
                iis_v[b][pl.ds(s, L)] = ii

            start_sc(b)
        return carry

    lax.fori_loop(0, NCHUNK // 2, _outer, 0)
    wait_sc(0)
    wait_sc(1)
    plsc.subcore_barrier()

    pltpu.sync_copy(acc_s.at[pl.ds(sid * SLICE, SLICE)], stage_v)
    pltpu.sync_copy(stage_v, out_hbm.at[cid, pl.ds(sid * SLICE, SLICE)])


def kernel(atomic_charges, distances, idx_i, idx_j):
    parts = _sc_energy(atomic_charges, distances, idx_i, idx_j)
    return parts[0, :N_NODES] + parts[1, :N_NODES]
